# Initial kernel scaffold; baseline (speedup 1.0000x reference)
import functools
import jax
import jax.numpy as jnp
from jax import lax
from jax.experimental import pallas as pl
from jax.experimental.pallas import tpu as pltpu
from jax.experimental.pallas import tpu_sc as plsc

N_TOK = 4096
D_MODEL = 1024
D_FF = 4096
N_EXPERTS = 16
TOP_K = 2
CAP = 640
NSLOT = N_EXPERTS * CAP
NA = TOP_K * N_TOK
TRASH = NSLOT

_SC_INFO = plsc.get_sparse_core_info()
NC, NS = _SC_INFO.num_cores, _SC_INFO.num_subcores
NW = NC * NS


def _router_body(x_ref, wr_ref, e0_ref, e1_ref, g0_ref, g1_ref):
    xb = x_ref[...]
    logits = jnp.dot(xb, wr_ref[...], preferred_element_type=jnp.float32)
    m = jnp.max(logits, axis=1, keepdims=True)
    p = jnp.exp(logits - m)
    probs = p / jnp.sum(p, axis=1, keepdims=True)
    iota = lax.broadcasted_iota(jnp.int32, probs.shape, 1)
    m1 = jnp.max(probs, axis=1, keepdims=True)
    i1 = jnp.min(jnp.where(probs >= m1, iota, N_EXPERTS), axis=1, keepdims=True)
    masked = jnp.where(iota == i1, -1.0, probs)
    m2 = jnp.max(masked, axis=1, keepdims=True)
    i2 = jnp.min(jnp.where(masked >= m2, iota, N_EXPERTS), axis=1, keepdims=True)
    s = m1 + m2 + 1e-6
    e0_ref[...] = i1.reshape(e0_ref.shape)
    e1_ref[...] = i2.reshape(e1_ref.shape)
    g0_ref[...] = (m1 / s).reshape(g0_ref.shape)
    g1_ref[...] = (m2 / s).reshape(g1_ref.shape)


def _router(x, w_router):
    BT = 512
    NB = N_TOK // BT
    outs = [
        jax.ShapeDtypeStruct((NB, 1, BT), jnp.int32),
        jax.ShapeDtypeStruct((NB, 1, BT), jnp.int32),
        jax.ShapeDtypeStruct((NB, 1, BT), jnp.float32),
        jax.ShapeDtypeStruct((NB, 1, BT), jnp.float32),
    ]
    ospec = pl.BlockSpec((1, 1, BT), lambda i: (i, 0, 0))
    return pl.pallas_call(
        _router_body,
        grid=(NB,),
        in_specs=[
            pl.BlockSpec((BT, D_MODEL), lambda i: (i, 0)),
            pl.BlockSpec((D_MODEL, N_EXPERTS), lambda i: (0, 0)),
        ],
        out_specs=[ospec, ospec, ospec, ospec],
        out_shape=outs,
    )(x, w_router)


def _slots_body(e0_ref, e1_ref, g0_ref, g1_ref, sd_ref, sc_ref, gf_ref):
    e01 = jnp.concatenate([e0_ref[...], e1_ref[...]], axis=1)
    g01 = jnp.concatenate([g0_ref[...], g1_ref[...]], axis=1)
    lanes = lax.broadcasted_iota(jnp.int32, (N_EXPERTS, NA), 0)
    oh = (jnp.broadcast_to(e01, (N_EXPERTS, NA)) == lanes).astype(jnp.int32)
    c = oh
    sh = 1
    while sh < NA:
        z = jnp.zeros((N_EXPERTS, sh), jnp.int32)
        c = c + jnp.concatenate([z, c[:, :-sh]], axis=1)
        sh *= 2
    pos = jnp.sum(c * oh, axis=0, keepdims=True) - 1
    valid = pos < CAP
    slot = e01 * CAP + jnp.minimum(pos, CAP - 1)
    sd_ref[...] = jnp.where(valid, slot, TRASH)
    sc_ref[...] = jnp.where(valid, slot, 0)
    gf_ref[...] = g01 * valid.astype(jnp.float32)


def _slots(e0, e1, g0, g1):
    outs = [
        jax.ShapeDtypeStruct((1, NA), jnp.int32),
        jax.ShapeDtypeStruct((1, NA), jnp.int32),
        jax.ShapeDtypeStruct((1, NA), jnp.float32),
    ]
    return pl.pallas_call(_slots_body, out_shape=outs)(e0, e1, g0, g1)


_D_CHUNK = 32
_D_PER_W = NA // NW


def _dispatch_body(x_hbm, slot_hbm, disp_hbm, rows_v, idx_v, sem):
    wid = lax.axis_index("s") * NC + lax.axis_index("c")
    base = wid * _D_PER_W
    tok_base = base % N_TOK
    for j in range(_D_PER_W // _D_CHUNK):
        off = j * _D_CHUNK
        pltpu.sync_copy(x_hbm.at[pl.ds(tok_base + off, _D_CHUNK)], rows_v)
        pltpu.sync_copy(slot_hbm.at[pl.ds(base + off, _D_CHUNK)], idx_v)
        pltpu.async_copy(rows_v, disp_hbm.at[idx_v], sem).wait()


def _dispatch(x, slot_d):
    mesh = plsc.VectorSubcoreMesh(core_axis_name="c", subcore_axis_name="s")
    kern = pl.kernel(
        _dispatch_body,
        out_type=jax.ShapeDtypeStruct((NSLOT + 8, D_MODEL), jnp.float32),
        mesh=mesh,
        scratch_types=[
            pltpu.VMEM((_D_CHUNK, D_MODEL), jnp.float32),
            pltpu.VMEM((_D_CHUNK,), jnp.int32),
            pltpu.SemaphoreType.DMA,
        ],
    )
    return kern(x, slot_d)


def _ffn_body(disp_ref, w1_ref, b1_ref, w2_ref, b2_ref, out_ref):
    f = pl.program_id(1)
    h = jnp.dot(disp_ref[...], w1_ref[0], preferred_element_type=jnp.float32)
    h = jax.nn.gelu(h + b1_ref[0])
    contrib = jnp.dot(h, w2_ref[0], preferred_element_type=jnp.float32)

    @pl.when(f == 0)
    def _():
        out_ref[...] = contrib + b2_ref[0]

    @pl.when(f != 0)
    def _():
        out_ref[...] = out_ref[...] + contrib


def _ffn(disp_pad, w1, b1, w2, b2, bf=1024):
    nf = D_FF // bf
    out = pl.pallas_call(
        _ffn_body,
        grid=(N_EXPERTS, nf),
        in_specs=[
            pl.BlockSpec((CAP, D_MODEL), lambda e, f: (e, 0)),
            pl.BlockSpec((1, D_MODEL, bf), lambda e, f: (e, 0, f)),
            pl.BlockSpec((1, 1, bf), lambda e, f: (e, 0, f)),
            pl.BlockSpec((1, bf, D_MODEL), lambda e, f: (e, f, 0)),
            pl.BlockSpec((1, 1, D_MODEL), lambda e, f: (e, 0, 0)),
        ],
        out_specs=pl.BlockSpec((1, CAP, D_MODEL), lambda e, f: (e, 0, 0)),
        out_shape=jax.ShapeDtypeStruct((N_EXPERTS, CAP, D_MODEL), jnp.float32),
        compiler_params=pltpu.CompilerParams(
            dimension_semantics=("arbitrary", "arbitrary"),
        ),
    )(disp_pad, w1, b1.reshape(N_EXPERTS, 1, D_FF), w2,
      b2.reshape(N_EXPERTS, 1, D_MODEL))
    return out.reshape(NSLOT, D_MODEL)


_C_CHUNK = 16
_C_PER_W = N_TOK // NW
_NCOL = D_MODEL // 16


def _combine_body(out_hbm, slot_hbm, g_hbm, y_hbm,
                  r0_v, r1_v, i0_v, i1_v, g0_v, g1_v, y_v, sem0, sem1):
    wid = lax.axis_index("s") * NC + lax.axis_index("c")
    base = wid * _C_PER_W
    for j in range(_C_PER_W // _C_CHUNK):
        t0 = base + j * _C_CHUNK
        pltpu.sync_copy(slot_hbm.at[pl.ds(t0, _C_CHUNK)], i0_v)
        pltpu.sync_copy(slot_hbm.at[pl.ds(N_TOK + t0, _C_CHUNK)], i1_v)
        pltpu.sync_copy(g_hbm.at[pl.ds(t0, _C_CHUNK)], g0_v)
        pltpu.sync_copy(g_hbm.at[pl.ds(N_TOK + t0, _C_CHUNK)], g1_v)
        cp0 = pltpu.async_copy(out_hbm.at[i0_v], r0_v, sem0)
        cp1 = pltpu.async_copy(out_hbm.at[i1_v], r1_v, sem1)
        cp0.wait()
        cp1.wait()
        g0 = g0_v[...]
        g1 = g1_v[...]
        for r in range(_C_CHUNK):
            g0s = g0[r]
            g1s = g1[r]

            def col(cc, _):
                v = (r0_v[r, pl.ds(cc * 16, 16)] * g0s
                     + r1_v[r, pl.ds(cc * 16, 16)] * g1s)
                y_v[r, pl.ds(cc * 16, 16)] = v
                return 0

            lax.fori_loop(0, _NCOL, col, 0, unroll=4)
        pltpu.sync_copy(y_v, y_hbm.at[pl.ds(t0, _C_CHUNK)])


def _combine(out_flat, slot_c, gfin):
    mesh = plsc.VectorSubcoreMesh(core_axis_name="c", subcore_axis_name="s")
    kern = pl.kernel(
        _combine_body,
        out_type=jax.ShapeDtypeStruct((N_TOK, D_MODEL), jnp.float32),
        mesh=mesh,
        scratch_types=[
            pltpu.VMEM((_C_CHUNK, D_MODEL), jnp.float32),
            pltpu.VMEM((_C_CHUNK, D_MODEL), jnp.float32),
            pltpu.VMEM((_C_CHUNK,), jnp.int32),
            pltpu.VMEM((_C_CHUNK,), jnp.int32),
            pltpu.VMEM((_C_CHUNK,), jnp.float32),
            pltpu.VMEM((_C_CHUNK,), jnp.float32),
            pltpu.VMEM((_C_CHUNK, D_MODEL), jnp.float32),
            pltpu.SemaphoreType.DMA,
            pltpu.SemaphoreType.DMA,
        ],
    )
    return kern(out_flat, slot_c, gfin)


def kernel(x, w_router, w1, b1, w2, b2):
    e0, e1, g0, g1 = _router(x, w_router)
    e0 = e0.reshape(1, N_TOK)
    e1 = e1.reshape(1, N_TOK)
    g0 = g0.reshape(1, N_TOK)
    g1 = g1.reshape(1, N_TOK)
    slot_d, slot_c, gfin = _slots(e0, e1, g0, g1)
    slot_d = slot_d.reshape(NA)
    slot_c = slot_c.reshape(NA)
    gfin = gfin.reshape(NA)
    disp_pad = _dispatch(x, slot_d)
    out_flat = _ffn(disp_pad, w1, b1, w2, b2)
    return _combine(out_flat, slot_c, gfin)


# speedup vs baseline: 1.4147x; 1.4147x over previous
import functools
import jax
import jax.numpy as jnp
from jax import lax
from jax.experimental import pallas as pl
from jax.experimental.pallas import tpu as pltpu
from jax.experimental.pallas import tpu_sc as plsc

N_TOK = 4096
D_MODEL = 1024
D_FF = 4096
N_EXPERTS = 16
TOP_K = 2
CAP = 640
NSLOT = N_EXPERTS * CAP
NA = TOP_K * N_TOK
TRASH = NSLOT

_SC_INFO = plsc.get_sparse_core_info()
NC, NS = _SC_INFO.num_cores, _SC_INFO.num_subcores
NW = NC * NS


def _router_body(x_ref, wr_ref, e0_ref, e1_ref, g0_ref, g1_ref):
    xb = x_ref[...]
    logits = jnp.dot(xb, wr_ref[...], preferred_element_type=jnp.float32)
    m = jnp.max(logits, axis=1, keepdims=True)
    p = jnp.exp(logits - m)
    probs = p / jnp.sum(p, axis=1, keepdims=True)
    iota = lax.broadcasted_iota(jnp.int32, probs.shape, 1)
    m1 = jnp.max(probs, axis=1, keepdims=True)
    i1 = jnp.min(jnp.where(probs >= m1, iota, N_EXPERTS), axis=1, keepdims=True)
    masked = jnp.where(iota == i1, -1.0, probs)
    m2 = jnp.max(masked, axis=1, keepdims=True)
    i2 = jnp.min(jnp.where(masked >= m2, iota, N_EXPERTS), axis=1, keepdims=True)
    s = m1 + m2 + 1e-6
    e0_ref[...] = i1.reshape(e0_ref.shape)
    e1_ref[...] = i2.reshape(e1_ref.shape)
    g0_ref[...] = (m1 / s).reshape(g0_ref.shape)
    g1_ref[...] = (m2 / s).reshape(g1_ref.shape)


def _router(x, w_router):
    BT = 512
    NB = N_TOK // BT
    outs = [
        jax.ShapeDtypeStruct((NB, 1, BT), jnp.int32),
        jax.ShapeDtypeStruct((NB, 1, BT), jnp.int32),
        jax.ShapeDtypeStruct((NB, 1, BT), jnp.float32),
        jax.ShapeDtypeStruct((NB, 1, BT), jnp.float32),
    ]
    ospec = pl.BlockSpec((1, 1, BT), lambda i: (i, 0, 0))
    return pl.pallas_call(
        _router_body,
        grid=(NB,),
        in_specs=[
            pl.BlockSpec((BT, D_MODEL), lambda i: (i, 0)),
            pl.BlockSpec((D_MODEL, N_EXPERTS), lambda i: (0, 0)),
        ],
        out_specs=[ospec, ospec, ospec, ospec],
        out_shape=outs,
    )(x, w_router)


def _slots_body(e0_ref, e1_ref, g0_ref, g1_ref, sd_ref, sc_ref, gf_ref):
    e01 = jnp.concatenate([e0_ref[...], e1_ref[...]], axis=1)
    g01 = jnp.concatenate([g0_ref[...], g1_ref[...]], axis=1)
    lanes = lax.broadcasted_iota(jnp.int32, (N_EXPERTS, NA), 0)
    oh = (jnp.broadcast_to(e01, (N_EXPERTS, NA)) == lanes).astype(jnp.int32)
    c = oh
    sh = 1
    while sh < NA:
        z = jnp.zeros((N_EXPERTS, sh), jnp.int32)
        c = c + jnp.concatenate([z, c[:, :-sh]], axis=1)
        sh *= 2
    pos = jnp.sum(c * oh, axis=0, keepdims=True) - 1
    valid = pos < CAP
    slot = e01 * CAP + jnp.minimum(pos, CAP - 1)
    sd_ref[...] = jnp.where(valid, slot, TRASH)
    sc_ref[...] = jnp.where(valid, slot, 0)
    gf_ref[...] = g01 * valid.astype(jnp.float32)


def _slots(e0, e1, g0, g1):
    outs = [
        jax.ShapeDtypeStruct((1, NA), jnp.int32),
        jax.ShapeDtypeStruct((1, NA), jnp.int32),
        jax.ShapeDtypeStruct((1, NA), jnp.float32),
    ]
    return pl.pallas_call(_slots_body, out_shape=outs)(e0, e1, g0, g1)


_D_CHUNK = 32
_D_PER_W = NA // NW


def _dispatch_body(x_hbm, slot_hbm, disp_hbm, rows_v, idx_v, sem):
    wid = lax.axis_index("s") * NC + lax.axis_index("c")
    base = wid * _D_PER_W
    tok_base = base % N_TOK
    for j in range(_D_PER_W // _D_CHUNK):
        off = j * _D_CHUNK
        pltpu.sync_copy(x_hbm.at[pl.ds(tok_base + off, _D_CHUNK)], rows_v)
        pltpu.sync_copy(slot_hbm.at[pl.ds(base + off, _D_CHUNK)], idx_v)
        pltpu.async_copy(rows_v, disp_hbm.at[idx_v], sem).wait()


def _dispatch(x, slot_d):
    mesh = plsc.VectorSubcoreMesh(core_axis_name="c", subcore_axis_name="s")
    kern = pl.kernel(
        _dispatch_body,
        out_type=jax.ShapeDtypeStruct((NSLOT + 8, D_MODEL), jnp.float32),
        mesh=mesh,
        scratch_types=[
            pltpu.VMEM((_D_CHUNK, D_MODEL), jnp.float32),
            pltpu.VMEM((_D_CHUNK,), jnp.int32),
            pltpu.SemaphoreType.DMA,
        ],
    )
    return kern(x, slot_d)


def _ffn_body(disp_ref, w1_ref, b1_ref, w2_ref, b2_ref, out_ref):
    f = pl.program_id(1)
    h = jnp.dot(disp_ref[...], w1_ref[0], preferred_element_type=jnp.float32)
    h = jax.nn.gelu(h + b1_ref[0])
    contrib = jnp.dot(h, w2_ref[0], preferred_element_type=jnp.float32)

    @pl.when(f == 0)
    def _():
        out_ref[0] = contrib + b2_ref[0]

    @pl.when(f != 0)
    def _():
        out_ref[0] = out_ref[0] + contrib


def _ffn(disp_pad, w1, b1, w2, b2, bf=1024):
    nf = D_FF // bf
    out = pl.pallas_call(
        _ffn_body,
        grid=(N_EXPERTS, nf),
        in_specs=[
            pl.BlockSpec((CAP, D_MODEL), lambda e, f: (e, 0)),
            pl.BlockSpec((1, D_MODEL, bf), lambda e, f: (e, 0, f)),
            pl.BlockSpec((1, 1, bf), lambda e, f: (e, 0, f)),
            pl.BlockSpec((1, bf, D_MODEL), lambda e, f: (e, f, 0)),
            pl.BlockSpec((1, 1, D_MODEL), lambda e, f: (e, 0, 0)),
        ],
        out_specs=pl.BlockSpec((1, CAP, D_MODEL), lambda e, f: (e, 0, 0)),
        out_shape=jax.ShapeDtypeStruct((N_EXPERTS, CAP, D_MODEL), jnp.float32),
        compiler_params=pltpu.CompilerParams(
            dimension_semantics=("arbitrary", "arbitrary"),
        ),
    )(disp_pad, w1, b1.reshape(N_EXPERTS, 1, D_FF), w2,
      b2.reshape(N_EXPERTS, 1, D_MODEL))
    return out.reshape(NSLOT, D_MODEL)


_C_CHUNK = 16
_C_PER_W = N_TOK // NW
_NCOL = D_MODEL // 16


def _combine_body(out_hbm, slot_hbm, g_hbm, y_hbm,
                  r0_v, r1_v, i0_v, i1_v, g0_v, g1_v, y_v, sem0, sem1):
    wid = lax.axis_index("s") * NC + lax.axis_index("c")
    base = wid * _C_PER_W
    for j in range(_C_PER_W // _C_CHUNK):
        t0 = base + j * _C_CHUNK
        pltpu.sync_copy(slot_hbm.at[pl.ds(t0, _C_CHUNK)], i0_v)
        pltpu.sync_copy(slot_hbm.at[pl.ds(N_TOK + t0, _C_CHUNK)], i1_v)
        pltpu.sync_copy(g_hbm.at[pl.ds(t0, _C_CHUNK)], g0_v)
        pltpu.sync_copy(g_hbm.at[pl.ds(N_TOK + t0, _C_CHUNK)], g1_v)
        cp0 = pltpu.async_copy(out_hbm.at[i0_v], r0_v, sem0)
        cp1 = pltpu.async_copy(out_hbm.at[i1_v], r1_v, sem1)
        cp0.wait()
        cp1.wait()
        g0 = g0_v[...]
        g1 = g1_v[...]
        for r in range(_C_CHUNK):
            g0s = g0[r]
            g1s = g1[r]

            def col(cc, _):
                v = (r0_v[r, pl.ds(cc * 16, 16)] * g0s
                     + r1_v[r, pl.ds(cc * 16, 16)] * g1s)
                y_v[r, pl.ds(cc * 16, 16)] = v
                return 0

            lax.fori_loop(0, _NCOL, col, 0, unroll=4)
        pltpu.sync_copy(y_v, y_hbm.at[pl.ds(t0, _C_CHUNK)])


def _combine(out_flat, slot_c, gfin):
    mesh = plsc.VectorSubcoreMesh(core_axis_name="c", subcore_axis_name="s")
    kern = pl.kernel(
        _combine_body,
        out_type=jax.ShapeDtypeStruct((N_TOK, D_MODEL), jnp.float32),
        mesh=mesh,
        scratch_types=[
            pltpu.VMEM((_C_CHUNK, D_MODEL), jnp.float32),
            pltpu.VMEM((_C_CHUNK, D_MODEL), jnp.float32),
            pltpu.VMEM((_C_CHUNK,), jnp.int32),
            pltpu.VMEM((_C_CHUNK,), jnp.int32),
            pltpu.VMEM((_C_CHUNK,), jnp.float32),
            pltpu.VMEM((_C_CHUNK,), jnp.float32),
            pltpu.VMEM((_C_CHUNK, D_MODEL), jnp.float32),
            pltpu.SemaphoreType.DMA,
            pltpu.SemaphoreType.DMA,
        ],
    )
    return kern(out_flat, slot_c, gfin)


def kernel(x, w_router, w1, b1, w2, b2):
    e0, e1, g0, g1 = _router(x, w_router)
    e0 = e0.reshape(1, N_TOK)
    e1 = e1.reshape(1, N_TOK)
    g0 = g0.reshape(1, N_TOK)
    g1 = g1.reshape(1, N_TOK)
    slot_d, slot_c, gfin = _slots(e0, e1, g0, g1)
    slot_d = slot_d.reshape(NA)
    slot_c = slot_c.reshape(NA)
    gfin = gfin.reshape(NA)
    disp_pad = _dispatch(x, slot_d)
    out_flat = _ffn(disp_pad, w1, b1, w2, b2)
    return _combine(out_flat, slot_c, gfin)
